# Initial kernel scaffold; baseline (speedup 1.0000x reference)
import jax
import jax.numpy as jnp
from jax.experimental import pallas as pl


def kernel(embeddings, ious, obj_labels, anc_labels, cls_labels, w, e):
    raise NotImplementedError("write your pallas kernel here")


# speedup vs baseline: 11.3240x; 11.3240x over previous
import jax
import jax.numpy as jnp
from jax.experimental import pallas as pl

ALPHA = 0.9
EPS = 1.19e-07


def _fused_kernel(emb_ref, obj_ref, anc_ref, w_ref, e_ref,
                  o_cls_ref, o_seq_ref, db_ref):
    x = emb_ref[:, :]
    xn = x * jax.lax.rsqrt(
        jnp.maximum(jnp.sum(x * x, axis=1, keepdims=True), 1e-12))

    w = w_ref[:, :]
    e = e_ref[:, :]
    wn = w * jax.lax.rsqrt(
        jnp.maximum(jnp.sum(w * w, axis=0, keepdims=True), 1e-12))
    en = e * jax.lax.rsqrt(
        jnp.maximum(jnp.sum(e * e, axis=0, keepdims=True), 1e-12))

    o_seq_ref[:, :] = jax.lax.dot(xn, wn, preferred_element_type=jnp.float32)
    o_cls_ref[:, :] = jax.lax.dot(xn, en, preferred_element_type=jnp.float32)

    n_classes = w.shape[1]
    labels = obj_ref[:, :]
    cls_iota = jax.lax.broadcasted_iota(jnp.int32, (1, n_classes), 1)
    oh = (labels == cls_iota).astype(jnp.float32)

    na = anc_ref[:, :]
    cnt = jnp.sum(oh, axis=0, keepdims=True)
    w_na = jnp.sum(oh * na, axis=0, keepdims=True)
    w_pa = jnp.sum(oh * jnp.abs(1.0 - na), axis=0, keepdims=True)

    pos_cls = jnp.clip(cnt, 0.0, 1.0)
    neg_cls = jnp.abs(1.0 - pos_cls)
    neg_anc = jnp.clip(w_na, 0.0, 1.0)
    pos_anc = jnp.clip(w_pa, 0.0, 1.0)

    masked = xn * na
    seg_sum = jax.lax.dot_general(
        masked, oh, (((0,), (0,)), ((), ())),
        preferred_element_type=jnp.float32)

    db_ref[:, :] = (ALPHA * w * neg_anc
                    + (1.0 - ALPHA) * seg_sum / (cnt + EPS)
                    + w * neg_cls + e * pos_anc)


def kernel(embeddings, ious, obj_labels, anc_labels, cls_labels, w, e):
    del ious, cls_labels
    B, N, D = embeddings.shape
    C = w.shape[1]
    S = B * N

    emb2 = embeddings.reshape(S, D).astype(jnp.float32)
    obj2 = obj_labels.reshape(S, 1).astype(jnp.int32)
    anc2 = anc_labels.reshape(S, 1).astype(jnp.float32)

    o_cls, o_seq, new_db = pl.pallas_call(
        _fused_kernel,
        out_shape=(
            jax.ShapeDtypeStruct((S, C), jnp.float32),
            jax.ShapeDtypeStruct((S, C), jnp.float32),
            jax.ShapeDtypeStruct((D, C), jnp.float32),
        ),
    )(emb2, obj2, anc2, w, e)

    return (o_cls.reshape(B, N, C), o_seq.reshape(B, N, C), new_db)
